# baseline (speedup 1.0000x reference)
import functools

import jax
import jax.numpy as jnp
from jax import lax
from jax.experimental import pallas as pl
from jax.experimental.pallas import tpu as pltpu
from jax.experimental.pallas import tpu_sc as plsc

EMB_DIM = 64
BATCH = 16384
NUM_CORES = 2
NUM_SUBCORES = 16
NUM_WORKERS = NUM_CORES * NUM_SUBCORES
B_PER_W = BATCH // NUM_WORKERS
GATHER_CHUNK = 128
N_CHUNKS = B_PER_W // GATHER_CHUNK


def _sc_kernel(users_hbm, items_hbm, uemb_hbm, iemb_hbm, out_hbm,
               idx_u, idx_i, rows_u, rows_i, out_v, sem):
  wid = lax.axis_index("s") * NUM_CORES + lax.axis_index("c")
  base = wid * B_PER_W

  pltpu.sync_copy(users_hbm.at[wid], idx_u)
  pltpu.sync_copy(items_hbm.at[wid], idx_i)

  copies = []
  for j in range(N_CHUNKS):
    copies.append(pltpu.async_copy(
        uemb_hbm.at[idx_u.at[j]],
        rows_u.at[pl.ds(j * GATHER_CHUNK, GATHER_CHUNK)], sem))
    copies.append(pltpu.async_copy(
        iemb_hbm.at[idx_i.at[j]],
        rows_i.at[pl.ds(j * GATHER_CHUNK, GATHER_CHUNK)], sem))
  for c in copies:
    c.wait()

  lane = lax.iota(jnp.int32, 16)

  @plsc.parallel_loop(0, B_PER_W // 16, 1, unroll=1)
  def _body(g):
    out_vec = jnp.zeros((16,), jnp.float32)
    for r in range(16):
      row = g * 16 + r
      acc = rows_u[row, pl.ds(0, 16)] * rows_i[row, pl.ds(0, 16)]
      for c in range(1, EMB_DIM // 16):
        acc += rows_u[row, pl.ds(c * 16, 16)] * rows_i[row, pl.ds(c * 16, 16)]
      s = jnp.full((16,), jnp.sum(acc))
      out_vec = jnp.where(lane == r, s, out_vec)
    out_v[pl.ds(g * 16, 16)] = out_vec

  pltpu.sync_copy(out_v, out_hbm.at[pl.ds(base, B_PER_W)])


@jax.jit
def kernel(users, items, user_emb, item_emb):
  users_2d = users.reshape(NUM_WORKERS, N_CHUNKS, GATHER_CHUNK)
  items_2d = items.reshape(NUM_WORKERS, N_CHUNKS, GATHER_CHUNK)
  mesh = plsc.VectorSubcoreMesh(core_axis_name="c", subcore_axis_name="s")
  f = pl.kernel(
      _sc_kernel,
      out_type=jax.ShapeDtypeStruct((BATCH,), jnp.float32),
      mesh=mesh,
      compiler_params=pltpu.CompilerParams(
          needs_layout_passes=False, use_tc_tiling_on_sc=False),
      scratch_types=[
          pltpu.VMEM((N_CHUNKS, GATHER_CHUNK), jnp.int32),
          pltpu.VMEM((N_CHUNKS, GATHER_CHUNK), jnp.int32),
          pltpu.VMEM((B_PER_W, EMB_DIM), jnp.float32),
          pltpu.VMEM((B_PER_W, EMB_DIM), jnp.float32),
          pltpu.VMEM((B_PER_W,), jnp.float32),
          pltpu.SemaphoreType.DMA,
      ],
  )
  return f(users_2d, items_2d, user_emb, item_emb)


# speedup vs baseline: 1.5643x; 1.5643x over previous
import jax
import jax.numpy as jnp
from jax import lax
from jax.experimental import pallas as pl
from jax.experimental.pallas import tpu as pltpu
from jax.experimental.pallas import tpu_sc as plsc

EMB_DIM = 64
BATCH = 16384
NUM_CORES = 2
NUM_SUBCORES = 16
NUM_WORKERS = NUM_CORES * NUM_SUBCORES
B_PER_W = BATCH // NUM_WORKERS
CHUNK = 16
N_CHUNKS = B_PER_W // CHUNK


def _sc_kernel(users_hbm, items_hbm, uemb_hbm, iemb_hbm, out_hbm,
               idx_u, idx_i, buf_u, buf_i, out_v,
               sem_u0, sem_i0, sem_u1, sem_i1):
  wid = lax.axis_index("s") * NUM_CORES + lax.axis_index("c")
  base = wid * B_PER_W

  pltpu.sync_copy(users_hbm.at[pl.ds(base, B_PER_W)], idx_u)
  pltpu.sync_copy(items_hbm.at[pl.ds(base, B_PER_W)], idx_i)

  sems = ((sem_u0, sem_i0), (sem_u1, sem_i1))
  lane = lax.iota(jnp.int32, 16)

  def fire(k, slot):
    uvec = idx_u[pl.ds(k * CHUNK, CHUNK)]
    ivec = idx_i[pl.ds(k * CHUNK, CHUNK)]
    su, si = sems[slot]
    for r in range(CHUNK):
      pltpu.async_copy(uemb_hbm.at[uvec[r]], buf_u.at[slot, r], su)
      pltpu.async_copy(iemb_hbm.at[ivec[r]], buf_i.at[slot, r], si)

  def wait(slot):
    su, si = sems[slot]
    for r in range(CHUNK):
      pltpu.make_async_copy(uemb_hbm.at[0], buf_u.at[slot, r], su).wait()
      pltpu.make_async_copy(iemb_hbm.at[0], buf_i.at[slot, r], si).wait()

  def compute(k, slot):
    out_vec = jnp.zeros((16,), jnp.float32)
    for r in range(CHUNK):
      acc = buf_u[slot, r, pl.ds(0, 16)] * buf_i[slot, r, pl.ds(0, 16)]
      for c in range(1, EMB_DIM // 16):
        acc += (buf_u[slot, r, pl.ds(c * 16, 16)] *
                buf_i[slot, r, pl.ds(c * 16, 16)])
      s = jnp.full((16,), jnp.sum(acc))
      out_vec = jnp.where(lane == r, s, out_vec)
    out_v[pl.ds(k * CHUNK, CHUNK)] = out_vec

  fire(0, 0)
  fire(1, 1)

  @pl.loop(0, N_CHUNKS - 2, step=2)
  def _main(k):
    wait(0)
    compute(k, 0)
    fire(k + 2, 0)
    wait(1)
    compute(k + 1, 1)
    fire(k + 3, 1)

  wait(0)
  compute(N_CHUNKS - 2, 0)
  wait(1)
  compute(N_CHUNKS - 1, 1)

  pltpu.sync_copy(out_v, out_hbm.at[pl.ds(base, B_PER_W)])


@jax.jit
def kernel(users, items, user_emb, item_emb):
  mesh = plsc.VectorSubcoreMesh(core_axis_name="c", subcore_axis_name="s")
  f = pl.kernel(
      _sc_kernel,
      out_type=jax.ShapeDtypeStruct((BATCH,), jnp.float32),
      mesh=mesh,
      compiler_params=pltpu.CompilerParams(
          needs_layout_passes=False, use_tc_tiling_on_sc=True),
      scratch_types=[
          pltpu.VMEM((B_PER_W,), jnp.int32),
          pltpu.VMEM((B_PER_W,), jnp.int32),
          pltpu.VMEM((2, CHUNK, EMB_DIM), jnp.float32),
          pltpu.VMEM((2, CHUNK, EMB_DIM), jnp.float32),
          pltpu.VMEM((B_PER_W,), jnp.float32),
          pltpu.SemaphoreType.DMA,
          pltpu.SemaphoreType.DMA,
          pltpu.SemaphoreType.DMA,
          pltpu.SemaphoreType.DMA,
      ],
  )
  return f(users, items, user_emb, item_emb)


# speedup vs baseline: 1.5794x; 1.0096x over previous
import jax
import jax.numpy as jnp
from jax import lax
from jax.experimental import pallas as pl
from jax.experimental.pallas import tpu as pltpu
from jax.experimental.pallas import tpu_sc as plsc

EMB_DIM = 64
BATCH = 16384
NUM_CORES = 2
NUM_SUBCORES = 16
NUM_WORKERS = NUM_CORES * NUM_SUBCORES
B_PER_W = BATCH // NUM_WORKERS
CHUNK = 16
N_CHUNKS = B_PER_W // CHUNK
NBUF = 4


def _sc_kernel(users_hbm, items_hbm, uemb_hbm, iemb_hbm, out_hbm,
               idx_u, idx_i, buf_u, buf_i, out_v,
               sem_u0, sem_u1, sem_u2, sem_u3,
               sem_i0, sem_i1, sem_i2, sem_i3):
  sem_u = (sem_u0, sem_u1, sem_u2, sem_u3)
  sem_i = (sem_i0, sem_i1, sem_i2, sem_i3)
  wid = lax.axis_index("s") * NUM_CORES + lax.axis_index("c")
  base = wid * B_PER_W

  pltpu.sync_copy(users_hbm.at[pl.ds(base, B_PER_W)], idx_u)
  pltpu.sync_copy(items_hbm.at[pl.ds(base, B_PER_W)], idx_i)

  lane = lax.iota(jnp.int32, 16)

  def fire(k, slot):
    uvec = idx_u[pl.ds(k * CHUNK, CHUNK)]
    ivec = idx_i[pl.ds(k * CHUNK, CHUNK)]
    for r in range(CHUNK):
      pltpu.async_copy(uemb_hbm.at[uvec[r]], buf_u.at[slot, r], sem_u[slot])
      pltpu.async_copy(iemb_hbm.at[ivec[r]], buf_i.at[slot, r], sem_i[slot])

  for b in range(NBUF - 1):
    fire(b, b)

  @pl.loop(0, N_CHUNKS, step=NBUF)
  def _body(g):
    for b in range(NBUF):
      k = g + b
      nxt = k + NBUF - 1

      @pl.when(nxt < N_CHUNKS)
      def _():
        fire(nxt, (b + NBUF - 1) % NBUF)

      pltpu.make_async_copy(
          uemb_hbm.at[pl.ds(0, CHUNK)], buf_u.at[b], sem_u[b]).wait()
      pltpu.make_async_copy(
          iemb_hbm.at[pl.ds(0, CHUNK)], buf_i.at[b], sem_i[b]).wait()

      out_vec = jnp.zeros((16,), jnp.float32)
      for r in range(CHUNK):
        acc = (buf_u[b, r, pl.ds(0, 16)] *
               buf_i[b, r, pl.ds(0, 16)])
        for c in range(1, EMB_DIM // 16):
          acc += (buf_u[b, r, pl.ds(c * 16, 16)] *
                  buf_i[b, r, pl.ds(c * 16, 16)])
        s = jnp.full((16,), jnp.sum(acc))
        out_vec = jnp.where(lane == r, s, out_vec)
      out_v[pl.ds(k * CHUNK, CHUNK)] = out_vec

  pltpu.sync_copy(out_v, out_hbm.at[pl.ds(base, B_PER_W)])


@jax.jit
def kernel(users, items, user_emb, item_emb):
  mesh = plsc.VectorSubcoreMesh(core_axis_name="c", subcore_axis_name="s")
  f = pl.kernel(
      _sc_kernel,
      out_type=jax.ShapeDtypeStruct((BATCH,), jnp.float32),
      mesh=mesh,
      compiler_params=pltpu.CompilerParams(
          needs_layout_passes=False, use_tc_tiling_on_sc=True),
      scratch_types=[
          pltpu.VMEM((B_PER_W,), jnp.int32),
          pltpu.VMEM((B_PER_W,), jnp.int32),
          pltpu.VMEM((NBUF, CHUNK, EMB_DIM), jnp.float32),
          pltpu.VMEM((NBUF, CHUNK, EMB_DIM), jnp.float32),
          pltpu.VMEM((B_PER_W,), jnp.float32),
          pltpu.SemaphoreType.DMA,
          pltpu.SemaphoreType.DMA,
          pltpu.SemaphoreType.DMA,
          pltpu.SemaphoreType.DMA,
          pltpu.SemaphoreType.DMA,
          pltpu.SemaphoreType.DMA,
          pltpu.SemaphoreType.DMA,
          pltpu.SemaphoreType.DMA,
      ],
  )
  return f(users, items, user_emb, item_emb)
